# baseline (speedup 1.0000x reference)
import jax
import jax.numpy as jnp
from jax.experimental import pallas as pl

B = 16384
D = 32
K = 64
ALPHA = 0.1
REG_W = 1e-5


def _partial_body(ut_ref, pt_ref, nt_ref, o_ref):
    ut = ut_ref[...]
    pt = pt_ref[...]
    nt = nt_ref[...]
    delta = jnp.sum(ut * (pt - nt), axis=0)
    sig = jnp.where(delta >= 0.0,
                    1.0 / (1.0 + jnp.exp(-delta)),
                    jnp.exp(delta) / (1.0 + jnp.exp(delta)))
    bpr_sum = jnp.sum(jnp.log(sig + 1e-8))
    uu = jnp.sum(ut * ut, axis=0)
    reg_sum = jnp.sum(uu) + jnp.sum(pt * pt) + jnp.sum(nt * nt)
    w = 1.0 / jnp.maximum(jnp.sqrt(uu), 1e-12)
    sumfeat = jnp.sum(ut * w[None, :], axis=1)
    sumfeatsq = jnp.sum(uu * w * w)
    o_ref[...] = jnp.concatenate(
        [jnp.reshape(bpr_sum, (1, 1)),
         jnp.reshape(reg_sum, (1, 1)),
         jnp.reshape(sumfeatsq, (1, 1)),
         jnp.reshape(sumfeat, (1, D)),
         jnp.zeros((1, 128 - 3 - D), jnp.float32)], axis=1)


def _combine_body(p1_ref, p2_ref, ct_ref, o_ref):
    ps = p1_ref[...] + p2_ref[...]
    bpr_sum = ps[:, 0:1]
    reg_sum = ps[:, 1:2]
    sumfeatsq = ps[:, 2:3]
    sumfeat = ps[:, 3:3 + D]
    ct = ct_ref[...]
    cw = 1.0 / jnp.maximum(jnp.sqrt(jnp.sum(ct * ct, axis=0)), 1e-12)
    cb = ct * cw[None, :]
    s_cc = jnp.sum(cb * cb)
    sum_cb = jnp.reshape(jnp.sum(cb, axis=1), (1, D))
    sdl = sumfeatsq / B + s_cc / K - (2.0 / (B * K)) * jnp.sum(sumfeat * sum_cb)
    com = (2.0 * K * s_cc - 2.0 * jnp.sum(sum_cb * sum_cb)) / (K * (K - 1))
    o_ref[...] = -bpr_sum / B + ALPHA * (sdl - com) + REG_W * reg_sum


def _rows(table, idx):
    dnums = jax.lax.GatherDimensionNumbers(
        offset_dims=(1,), collapsed_slice_dims=(0,), start_index_map=(0,))
    return jax.lax.gather(
        table, idx[:, None], dnums, slice_sizes=(1, table.shape[1]),
        mode=jax.lax.GatherScatterMode.PROMISE_IN_BOUNDS)


def _partial(u, p, n):
    return pl.pallas_call(
        _partial_body,
        out_shape=jax.ShapeDtypeStruct((1, 128), jnp.float32),
    )(u.T, p.T, n.T)


def kernel(uid, src_ids, pos_ids, neg_ids, src_user_emb, src_item_emb, src_clusters):
    del src_ids
    h = B // 2
    parts = []
    for sl in (slice(0, h), slice(h, B)):
        u = _rows(src_user_emb, uid[sl])
        p = _rows(src_item_emb, pos_ids[sl])
        n = _rows(src_item_emb, neg_ids[sl])
        parts.append(_partial(u, p, n))
    out = pl.pallas_call(
        _combine_body,
        out_shape=jax.ShapeDtypeStruct((1, 1), jnp.float32),
    )(parts[0], parts[1], src_clusters.T)
    return out[0, 0]


# speedup vs baseline: 1.2473x; 1.2473x over previous
import jax
import jax.numpy as jnp
from jax.experimental import pallas as pl

B = 16384
D = 32
K = 64
ALPHA = 0.1
REG_W = 1e-5


def _loss_body(ut_ref, pnt_ref, ct_ref, o_ref):
    ut = ut_ref[...]
    pt = pnt_ref[:, :B]
    nt = pnt_ref[:, B:]
    ct = ct_ref[...]
    delta = jnp.sum(ut * (pt - nt), axis=0, keepdims=True)
    sig = jnp.where(delta >= 0.0,
                    1.0 / (1.0 + jnp.exp(-delta)),
                    jnp.exp(delta) / (1.0 + jnp.exp(delta)))
    bpr_sum = jnp.sum(jnp.log(sig + 1e-8))
    uu = jnp.sum(ut * ut, axis=0, keepdims=True)
    reg_sum = jnp.sum(uu) + jnp.sum(pt * pt) + jnp.sum(nt * nt)
    w = 1.0 / jnp.maximum(jnp.sqrt(uu), 1e-12)
    sumfeat = jnp.sum(ut * w, axis=1, keepdims=True)
    sumfeatsq = jnp.sum(uu * w * w)
    cw = 1.0 / jnp.maximum(jnp.sqrt(jnp.sum(ct * ct, axis=0, keepdims=True)),
                           1e-12)
    cb = ct * cw
    s_cc = jnp.sum(cb * cb)
    sum_cb = jnp.sum(cb, axis=1, keepdims=True)
    sdl = sumfeatsq / B + s_cc / K - (2.0 / (B * K)) * jnp.sum(sumfeat * sum_cb)
    com = (2.0 * K * s_cc - 2.0 * jnp.sum(sum_cb * sum_cb)) / (K * (K - 1))
    total = -bpr_sum / B + ALPHA * (sdl - com) + REG_W * reg_sum
    o_ref[...] = jnp.reshape(total, (1, 1))


def _rows(table, idx):
    dnums = jax.lax.GatherDimensionNumbers(
        offset_dims=(1,), collapsed_slice_dims=(0,), start_index_map=(0,))
    return jax.lax.gather(
        table, idx[:, None], dnums, slice_sizes=(1, table.shape[1]),
        mode=jax.lax.GatherScatterMode.PROMISE_IN_BOUNDS)


def kernel(uid, src_ids, pos_ids, neg_ids, src_user_emb, src_item_emb, src_clusters):
    del src_ids
    pn = _rows(src_item_emb, jnp.concatenate([pos_ids, neg_ids]))
    u = _rows(src_user_emb, uid)
    out = pl.pallas_call(
        _loss_body,
        out_shape=jax.ShapeDtypeStruct((1, 1), jnp.float32),
    )(u.T, pn.T, src_clusters.T)
    return out[0, 0]


# speedup vs baseline: 1.2536x; 1.0050x over previous
import jax
import jax.numpy as jnp
from jax.experimental import pallas as pl

B = 16384
D = 32
K = 64
ALPHA = 0.1
REG_W = 1e-5


def _ustats_body(ut_ref, o_ref):
    ut = ut_ref[...]
    uu = jnp.sum(ut * ut, axis=0, keepdims=True)
    reg_u = jnp.sum(uu)
    w = 1.0 / jnp.maximum(jnp.sqrt(uu), 1e-12)
    sumfeat = jnp.sum(ut * w, axis=1, keepdims=True)
    sumfeatsq = jnp.sum(uu * w * w)
    o_ref[...] = jnp.concatenate(
        [jnp.reshape(reg_u, (1, 1)),
         jnp.reshape(sumfeatsq, (1, 1)),
         jnp.reshape(sumfeat, (1, D)),
         jnp.zeros((1, 128 - 2 - D), jnp.float32)], axis=1)


def _final_body(ut_ref, pnt_ref, us_ref, ct_ref, o_ref):
    ut = ut_ref[...]
    pt = pnt_ref[:, :B]
    nt = pnt_ref[:, B:]
    us = us_ref[...]
    ct = ct_ref[...]
    delta = jnp.sum(ut * (pt - nt), axis=0, keepdims=True)
    sig = jnp.where(delta >= 0.0,
                    1.0 / (1.0 + jnp.exp(-delta)),
                    jnp.exp(delta) / (1.0 + jnp.exp(delta)))
    bpr_sum = jnp.sum(jnp.log(sig + 1e-8))
    reg_sum = us[:, 0:1] + jnp.sum(pt * pt) + jnp.sum(nt * nt)
    sumfeatsq = us[:, 1:2]
    sumfeat = us[:, 2:2 + D]
    cw = 1.0 / jnp.maximum(jnp.sqrt(jnp.sum(ct * ct, axis=0, keepdims=True)),
                           1e-12)
    cb = ct * cw
    s_cc = jnp.sum(cb * cb)
    sum_cb = jnp.reshape(jnp.sum(cb, axis=1, keepdims=True), (1, D))
    sdl = sumfeatsq / B + s_cc / K - (2.0 / (B * K)) * jnp.sum(sumfeat * sum_cb)
    com = (2.0 * K * s_cc - 2.0 * jnp.sum(sum_cb * sum_cb)) / (K * (K - 1))
    o_ref[...] = -bpr_sum / B + ALPHA * (sdl - com) + REG_W * reg_sum


def _rows(table, idx):
    dnums = jax.lax.GatherDimensionNumbers(
        offset_dims=(1,), collapsed_slice_dims=(0,), start_index_map=(0,))
    return jax.lax.gather(
        table, idx[:, None], dnums, slice_sizes=(1, table.shape[1]),
        mode=jax.lax.GatherScatterMode.PROMISE_IN_BOUNDS)


def kernel(uid, src_ids, pos_ids, neg_ids, src_user_emb, src_item_emb, src_clusters):
    del src_ids
    pn = _rows(src_item_emb, jnp.concatenate([pos_ids, neg_ids]))
    u = _rows(src_user_emb, uid)
    ut = u.T
    ustats = pl.pallas_call(
        _ustats_body,
        out_shape=jax.ShapeDtypeStruct((1, 128), jnp.float32),
    )(ut)
    out = pl.pallas_call(
        _final_body,
        out_shape=jax.ShapeDtypeStruct((1, 1), jnp.float32),
    )(ut, pn.T, ustats, src_clusters.T)
    return out[0, 0]
